# baseline (speedup 1.0000x reference)
import functools

import jax
import jax.numpy as jnp
from jax import lax
from jax.experimental import pallas as pl
from jax.experimental.pallas import tpu as pltpu
from jax.experimental.pallas import tpu_sc as plsc

_N_ROWS = 16384
_N_CLASSES = 1000
_EPS = 0.001
_ON = 1.0 / _N_CLASSES + _EPS
_OFF = 1.0 / _N_CLASSES - _EPS / (_N_CLASSES - 1)

_TC_ROWS = 8192
_SC_ROWS = _N_ROWS - _TC_ROWS

_BLOCK_ROWS = 2048

_N_WORKERS = 32
_ROWS_PER_WORKER = _SC_ROWS // _N_WORKERS
_BATCH = 32
_N_BATCHES = _ROWS_PER_WORKER // _BATCH
_FULL_CHUNKS = _N_CLASSES // 16
_TAIL_OFF = _N_CLASSES - 16


def _tc_body(x_ref, o_ref):
    x = x_ref[...]
    rowmax = jnp.max(x, axis=1, keepdims=True)
    cols = lax.broadcasted_iota(jnp.int32, x.shape, 1)
    big = jnp.int32(_N_CLASSES)
    amax = jnp.min(jnp.where(x == rowmax, cols, big), axis=1, keepdims=True)
    o_ref[...] = jnp.where(cols == amax, jnp.float32(_ON), jnp.float32(_OFF))


def _tc_part(x):
    grid = _TC_ROWS // _BLOCK_ROWS
    return pl.pallas_call(
        _tc_body,
        grid=(grid,),
        in_specs=[pl.BlockSpec((_BLOCK_ROWS, _N_CLASSES), lambda i: (i, 0))],
        out_specs=pl.BlockSpec((_BLOCK_ROWS, _N_CLASSES), lambda i: (i, 0)),
        out_shape=jax.ShapeDtypeStruct((_TC_ROWS, _N_CLASSES), jnp.float32),
    )(x)


def _sc_body(in_hbm, out_hbm, in0, in1, ob0, ob1, si0, si1, so0, so1):
    wid = lax.axis_index("s") * 2 + lax.axis_index("c")
    row0 = wid * _ROWS_PER_WORKER
    in_row0 = _TC_ROWS + row0

    lane = lax.iota(jnp.int32, 16)
    off_vec = jnp.full((16,), _OFF, jnp.float32)
    on_vec = jnp.full((16,), _ON, jnp.float32)
    ninf = jnp.full((16,), -jnp.inf, jnp.float32)
    zeros_i = jnp.zeros((16,), jnp.int32)

    inbufs = (in0, in1)
    outbufs = (ob0, ob1)
    isems = (si0, si1)
    osems = (so0, so1)

    for ob in outbufs:
        def _fill_row(r, _, ob=ob):
            for c in range(_FULL_CHUNKS):
                ob[r, pl.ds(c * 16, 16)] = off_vec
            ob[r, pl.ds(_TAIL_OFF, 16)] = off_vec
            return _
        lax.fori_loop(0, _BATCH, _fill_row, None)

    base_k = tuple(lane + 16 * k for k in range(4))
    ones_i = jnp.ones((16,), jnp.int32)
    big_i = jnp.full((16,), _N_CLASSES, jnp.int32)

    def _merge(mv_a, ci_a, mv_b, ci_b):
        take_b = (mv_b > mv_a) | ((mv_b == mv_a) & (ci_b < ci_a))
        return jnp.where(take_b, mv_b, mv_a), jnp.where(take_b, ci_b, ci_a)

    def _argmax_group(inb, g):
        def _row(r, acc):
            rr = g * 16 + r

            def _step(t, carry):
                tv, mv0, mi0, mv1, mi1, mv2, mi2, mv3, mi3 = carry
                o = t * 64
                x0 = inb[rr, pl.ds(o, 16)]
                x1 = inb[rr, pl.ds(o + 16, 16)]
                x2 = inb[rr, pl.ds(o + 32, 16)]
                x3 = inb[rr, pl.ds(o + 48, 16)]
                g0 = x0 > mv0
                g1 = x1 > mv1
                g2 = x2 > mv2
                g3 = x3 > mv3
                return (tv + ones_i,
                        jnp.where(g0, x0, mv0), jnp.where(g0, tv, mi0),
                        jnp.where(g1, x1, mv1), jnp.where(g1, tv, mi1),
                        jnp.where(g2, x2, mv2), jnp.where(g2, tv, mi2),
                        jnp.where(g3, x3, mv3), jnp.where(g3, tv, mi3))

            init = (zeros_i,
                    ninf, zeros_i, ninf, zeros_i,
                    ninf, zeros_i, ninf, zeros_i)
            _, mv0, mi0, mv1, mi1, mv2, mi2, mv3, mi3 = lax.fori_loop(
                0, 15, _step, init)

            c0 = (mi0 << 6) + base_k[0]
            c1 = (mi1 << 6) + base_k[1]
            c2 = (mi2 << 6) + base_k[2]
            c3 = (mi3 << 6) + base_k[3]
            mva, cia = _merge(mv0, c0, mv1, c1)
            mvb, cib = _merge(mv2, c2, mv3, c3)
            mv, ci = _merge(mva, cia, mvb, cib)

            for off in (960, 976, _TAIL_OFF):
                x = inb[rr, pl.ds(off, 16)]
                gt = x > mv
                mv = jnp.where(gt, x, mv)
                ci = jnp.where(gt, off + lane, ci)

            m = jnp.max(mv)
            a = jnp.min(jnp.where(mv == m, ci, big_i))
            return jnp.where(lane == r, a, acc)

        return lax.fori_loop(0, 16, _row, zeros_i)

    in_handles = {0: pltpu.async_copy(
        in_hbm.at[pl.ds(in_row0, _BATCH)], inbufs[0], isems[0])}
    out_handles = {}
    restore_pos = {}

    for b in range(_N_BATCHES):
        p = b & 1
        if b + 1 < _N_BATCHES:
            in_handles[b + 1] = pltpu.async_copy(
                in_hbm.at[pl.ds(in_row0 + (b + 1) * _BATCH, _BATCH)],
                inbufs[(b + 1) & 1], isems[(b + 1) & 1])
        in_handles[b].wait()

        if b >= 2:
            out_handles[b - 2].wait()
            for rows, cols in restore_pos[p]:
                plsc.store_scatter(outbufs[p], [rows, cols], off_vec)

        pos = []
        for g in range(_BATCH // 16):
            cols = _argmax_group(inbufs[p], g)
            rows = g * 16 + lane
            plsc.store_scatter(outbufs[p], [rows, cols], on_vec)
            pos.append((rows, cols))
        restore_pos[p] = pos

        out_handles[b] = pltpu.async_copy(
            outbufs[p],
            out_hbm.at[pl.ds(row0 + b * _BATCH, _BATCH)], osems[p])

    out_handles[_N_BATCHES - 2].wait()
    out_handles[_N_BATCHES - 1].wait()


def _sc_part(x):
    mesh = plsc.VectorSubcoreMesh(core_axis_name="c", subcore_axis_name="s")
    fn = functools.partial(
        pl.kernel,
        out_type=jax.ShapeDtypeStruct((_SC_ROWS, _N_CLASSES), jnp.float32),
        mesh=mesh,
        scratch_types=[
            pltpu.VMEM((_BATCH, _N_CLASSES), jnp.float32),
            pltpu.VMEM((_BATCH, _N_CLASSES), jnp.float32),
            pltpu.VMEM((_BATCH, _N_CLASSES), jnp.float32),
            pltpu.VMEM((_BATCH, _N_CLASSES), jnp.float32),
            pltpu.SemaphoreType.DMA,
            pltpu.SemaphoreType.DMA,
            pltpu.SemaphoreType.DMA,
            pltpu.SemaphoreType.DMA,
        ],
        compiler_params=pltpu.CompilerParams(needs_layout_passes=False),
    )(_sc_body)
    return fn(x)


def kernel(input):
    top = _tc_part(input)
    bottom = _sc_part(input)
    return jnp.concatenate([top, bottom], axis=0)


# speedup vs baseline: 1.1059x; 1.1059x over previous
import functools

import jax
import jax.numpy as jnp
from jax import lax
from jax.experimental import pallas as pl
from jax.experimental.pallas import tpu as pltpu
from jax.experimental.pallas import tpu_sc as plsc

_N_ROWS = 16384
_N_CLASSES = 1000
_EPS = 0.001
_ON = 1.0 / _N_CLASSES + _EPS
_OFF = 1.0 / _N_CLASSES - _EPS / (_N_CLASSES - 1)

_N_WORKERS = 32
_ROWS_PER_WORKER = _N_ROWS // _N_WORKERS
_BATCH = 32
_N_BATCHES = _ROWS_PER_WORKER // _BATCH
_N_PAIRS = _N_BATCHES // 2
_FULL_CHUNKS = _N_CLASSES // 16
_TAIL_OFF = _N_CLASSES - 16


def _sc_body(in_hbm, out_hbm, in0, in1, ob0, ob1, si0, si1, so0, so1):
    wid = lax.axis_index("s") * 2 + lax.axis_index("c")
    row0 = wid * _ROWS_PER_WORKER

    lane = lax.iota(jnp.int32, 16)
    lane16 = lane + 16
    off_vec = jnp.full((16,), _OFF, jnp.float32)
    on_vec = jnp.full((16,), _ON, jnp.float32)
    ninf = jnp.full((16,), -jnp.inf, jnp.float32)
    zeros_i = jnp.zeros((16,), jnp.int32)

    base_k = tuple(lane + 16 * k for k in range(4))
    ones_i = jnp.ones((16,), jnp.int32)
    big_i = jnp.full((16,), _N_CLASSES, jnp.int32)

    def _merge(mv_a, ci_a, mv_b, ci_b):
        take_b = (mv_b > mv_a) | ((mv_b == mv_a) & (ci_b < ci_a))
        return jnp.where(take_b, mv_b, mv_a), jnp.where(take_b, ci_b, ci_a)

    def _argmax_group(inb, g):
        def _row(r, acc):
            rr = g * 16 + r

            def _step(t, carry):
                tv, mv0, mi0, mv1, mi1, mv2, mi2, mv3, mi3 = carry
                o = t * 64
                x0 = inb[rr, pl.ds(o, 16)]
                x1 = inb[rr, pl.ds(o + 16, 16)]
                x2 = inb[rr, pl.ds(o + 32, 16)]
                x3 = inb[rr, pl.ds(o + 48, 16)]
                g0 = x0 > mv0
                g1 = x1 > mv1
                g2 = x2 > mv2
                g3 = x3 > mv3
                return (tv + ones_i,
                        jnp.where(g0, x0, mv0), jnp.where(g0, tv, mi0),
                        jnp.where(g1, x1, mv1), jnp.where(g1, tv, mi1),
                        jnp.where(g2, x2, mv2), jnp.where(g2, tv, mi2),
                        jnp.where(g3, x3, mv3), jnp.where(g3, tv, mi3))

            init = (zeros_i,
                    ninf, zeros_i, ninf, zeros_i,
                    ninf, zeros_i, ninf, zeros_i)
            _, mv0, mi0, mv1, mi1, mv2, mi2, mv3, mi3 = lax.fori_loop(
                0, 15, _step, init)

            c0 = (mi0 << 6) + base_k[0]
            c1 = (mi1 << 6) + base_k[1]
            c2 = (mi2 << 6) + base_k[2]
            c3 = (mi3 << 6) + base_k[3]
            mva, cia = _merge(mv0, c0, mv1, c1)
            mvb, cib = _merge(mv2, c2, mv3, c3)
            mv, ci = _merge(mva, cia, mvb, cib)

            for off in (960, 976, _TAIL_OFF):
                x = inb[rr, pl.ds(off, 16)]
                gt = x > mv
                mv = jnp.where(gt, x, mv)
                ci = jnp.where(gt, off + lane, ci)

            m = jnp.max(mv)
            a = jnp.min(jnp.where(mv == m, ci, big_i))
            return jnp.where(lane == r, a, acc)

        return lax.fori_loop(0, 16, _row, zeros_i)

    pltpu.async_copy(in_hbm.at[pl.ds(row0, _BATCH)], in0, si0)
    pltpu.async_copy(in_hbm.at[pl.ds(row0 + _BATCH, _BATCH)], in1, si1)

    for ob in (ob0, ob1):
        def _fill_row(r, _, ob=ob):
            for c in range(_FULL_CHUNKS):
                ob[r, pl.ds(c * 16, 16)] = off_vec
            ob[r, pl.ds(_TAIL_OFF, 16)] = off_vec
            return _
        lax.fori_loop(0, _BATCH, _fill_row, None)

    pltpu.async_copy(ob0, out_hbm.at[pl.ds(row0, _BATCH)], so0)
    pltpu.async_copy(ob1, out_hbm.at[pl.ds(row0 + _BATCH, _BATCH)], so1)

    def _half(t, inb, ob, isem, osem, b, pc_g0, pc_g1):
        cur = row0 + b * _BATCH
        prev = row0 + jnp.maximum(b - 2, 0) * _BATCH
        nxt = row0 + jnp.minimum(b + 2, _N_BATCHES - 1) * _BATCH

        pltpu.make_async_copy(ob, out_hbm.at[pl.ds(prev, _BATCH)], osem).wait()
        plsc.store_scatter(ob, [lane, pc_g0], off_vec)
        plsc.store_scatter(ob, [lane16, pc_g1], off_vec)

        pltpu.make_async_copy(in_hbm.at[pl.ds(cur, _BATCH)], inb, isem).wait()
        cols_g0 = _argmax_group(inb, 0)
        cols_g1 = _argmax_group(inb, 1)
        pltpu.async_copy(in_hbm.at[pl.ds(nxt, _BATCH)], inb, isem)

        plsc.store_scatter(ob, [lane, cols_g0], on_vec)
        plsc.store_scatter(ob, [lane16, cols_g1], on_vec)
        pltpu.async_copy(ob, out_hbm.at[pl.ds(cur, _BATCH)], osem)
        return cols_g0, cols_g1

    def _pair(t, carry):
        pc00, pc01, pc10, pc11 = carry
        pc00, pc01 = _half(t, in0, ob0, si0, so0, 2 * t, pc00, pc01)
        pc10, pc11 = _half(t, in1, ob1, si1, so1, 2 * t + 1, pc10, pc11)
        return pc00, pc01, pc10, pc11

    lax.fori_loop(0, _N_PAIRS, _pair,
                  (zeros_i, zeros_i, zeros_i, zeros_i))

    last0 = row0 + (_N_BATCHES - 2) * _BATCH
    last1 = row0 + (_N_BATCHES - 1) * _BATCH
    pltpu.make_async_copy(ob0, out_hbm.at[pl.ds(last0, _BATCH)], so0).wait()
    pltpu.make_async_copy(ob1, out_hbm.at[pl.ds(last1, _BATCH)], so1).wait()
    pltpu.make_async_copy(in_hbm.at[pl.ds(last1, _BATCH)], in0, si0).wait()
    pltpu.make_async_copy(in_hbm.at[pl.ds(last1, _BATCH)], in1, si1).wait()


def kernel(input):
    mesh = plsc.VectorSubcoreMesh(core_axis_name="c", subcore_axis_name="s")
    fn = functools.partial(
        pl.kernel,
        out_type=jax.ShapeDtypeStruct((_N_ROWS, _N_CLASSES), jnp.float32),
        mesh=mesh,
        scratch_types=[
            pltpu.VMEM((_BATCH, _N_CLASSES), jnp.float32),
            pltpu.VMEM((_BATCH, _N_CLASSES), jnp.float32),
            pltpu.VMEM((_BATCH, _N_CLASSES), jnp.float32),
            pltpu.VMEM((_BATCH, _N_CLASSES), jnp.float32),
            pltpu.SemaphoreType.DMA,
            pltpu.SemaphoreType.DMA,
            pltpu.SemaphoreType.DMA,
            pltpu.SemaphoreType.DMA,
        ],
        compiler_params=pltpu.CompilerParams(needs_layout_passes=False),
    )(_sc_body)
    return fn(input)


# speedup vs baseline: 1.1215x; 1.0141x over previous
import functools

import jax
import jax.numpy as jnp
from jax import lax
from jax.experimental import pallas as pl
from jax.experimental.pallas import tpu as pltpu
from jax.experimental.pallas import tpu_sc as plsc

_N_ROWS = 16384
_N_CLASSES = 1000
_EPS = 0.001
_ON = 1.0 / _N_CLASSES + _EPS
_OFF = 1.0 / _N_CLASSES - _EPS / (_N_CLASSES - 1)

_N_WORKERS = 32
_ROWS_PER_WORKER = _N_ROWS // _N_WORKERS
_BATCH = 16
_N_BATCHES = _ROWS_PER_WORKER // _BATCH
_N_PAR = 4
_N_STEPS = _N_BATCHES // _N_PAR
_FULL_CHUNKS = _N_CLASSES // 16
_TAIL_OFF = _N_CLASSES - 16


def _sc_body(in_hbm, out_hbm,
             in0, in1, in2, in3, ob0, ob1, ob2, ob3,
             si0, si1, si2, si3, so0, so1, so2, so3):
    wid = lax.axis_index("s") * 2 + lax.axis_index("c")
    row0 = wid * _ROWS_PER_WORKER

    inbufs = (in0, in1, in2, in3)
    outbufs = (ob0, ob1, ob2, ob3)
    isems = (si0, si1, si2, si3)
    osems = (so0, so1, so2, so3)

    lane = lax.iota(jnp.int32, 16)
    off_vec = jnp.full((16,), _OFF, jnp.float32)
    on_vec = jnp.full((16,), _ON, jnp.float32)
    ninf = jnp.full((16,), -jnp.inf, jnp.float32)
    zeros_i = jnp.zeros((16,), jnp.int32)

    base_k = tuple(lane + 16 * k for k in range(4))
    ones_i = jnp.ones((16,), jnp.int32)
    big_i = jnp.full((16,), _N_CLASSES, jnp.int32)

    def _merge(mv_a, ci_a, mv_b, ci_b):
        take_b = (mv_b > mv_a) | ((mv_b == mv_a) & (ci_b < ci_a))
        return jnp.where(take_b, mv_b, mv_a), jnp.where(take_b, ci_b, ci_a)

    def _argmax_group(inb, g):
        def _row(r, acc):
            rr = g * 16 + r

            def _step(t, carry):
                tv, mv0, mi0, mv1, mi1, mv2, mi2, mv3, mi3 = carry
                o = t * 64
                x0 = inb[rr, pl.ds(o, 16)]
                x1 = inb[rr, pl.ds(o + 16, 16)]
                x2 = inb[rr, pl.ds(o + 32, 16)]
                x3 = inb[rr, pl.ds(o + 48, 16)]
                g0 = x0 > mv0
                g1 = x1 > mv1
                g2 = x2 > mv2
                g3 = x3 > mv3
                return (tv + ones_i,
                        jnp.where(g0, x0, mv0), jnp.where(g0, tv, mi0),
                        jnp.where(g1, x1, mv1), jnp.where(g1, tv, mi1),
                        jnp.where(g2, x2, mv2), jnp.where(g2, tv, mi2),
                        jnp.where(g3, x3, mv3), jnp.where(g3, tv, mi3))

            init = (zeros_i,
                    ninf, zeros_i, ninf, zeros_i,
                    ninf, zeros_i, ninf, zeros_i)
            _, mv0, mi0, mv1, mi1, mv2, mi2, mv3, mi3 = lax.fori_loop(
                0, 15, _step, init)

            c0 = (mi0 << 6) + base_k[0]
            c1 = (mi1 << 6) + base_k[1]
            c2 = (mi2 << 6) + base_k[2]
            c3 = (mi3 << 6) + base_k[3]
            mva, cia = _merge(mv0, c0, mv1, c1)
            mvb, cib = _merge(mv2, c2, mv3, c3)
            mv, ci = _merge(mva, cia, mvb, cib)

            for off in (960, 976, _TAIL_OFF):
                x = inb[rr, pl.ds(off, 16)]
                gt = x > mv
                mv = jnp.where(gt, x, mv)
                ci = jnp.where(gt, off + lane, ci)

            m = jnp.max(mv)
            a = jnp.min(jnp.where(mv == m, ci, big_i))
            return jnp.where(lane == r, a, acc)

        return lax.fori_loop(0, 16, _row, zeros_i)

    for j in range(_N_PAR):
        pltpu.async_copy(
            in_hbm.at[pl.ds(row0 + j * _BATCH, _BATCH)], inbufs[j], isems[j])

    for ob in outbufs:
        def _fill_row(r, _, ob=ob):
            for c in range(_FULL_CHUNKS):
                ob[r, pl.ds(c * 16, 16)] = off_vec
            ob[r, pl.ds(_TAIL_OFF, 16)] = off_vec
            return _
        lax.fori_loop(0, _BATCH, _fill_row, None)

    for j in range(_N_PAR):
        pltpu.async_copy(
            outbufs[j], out_hbm.at[pl.ds(row0 + j * _BATCH, _BATCH)], osems[j])

    def _one(inb, ob, isem, osem, b, pc):
        cur = row0 + b * _BATCH
        prev = row0 + jnp.maximum(b - _N_PAR, 0) * _BATCH
        nxt = row0 + jnp.minimum(b + _N_PAR, _N_BATCHES - 1) * _BATCH

        pltpu.make_async_copy(ob, out_hbm.at[pl.ds(prev, _BATCH)], osem).wait()
        plsc.store_scatter(ob, [lane, pc], off_vec)

        pltpu.make_async_copy(in_hbm.at[pl.ds(cur, _BATCH)], inb, isem).wait()
        cols = _argmax_group(inb, 0)
        pltpu.async_copy(in_hbm.at[pl.ds(nxt, _BATCH)], inb, isem)

        plsc.store_scatter(ob, [lane, cols], on_vec)
        pltpu.async_copy(ob, out_hbm.at[pl.ds(cur, _BATCH)], osem)
        return cols

    def _step4(t, carry):
        return tuple(
            _one(inbufs[j], outbufs[j], isems[j], osems[j],
                 _N_PAR * t + j, carry[j])
            for j in range(_N_PAR))

    lax.fori_loop(0, _N_STEPS, _step4, (zeros_i,) * _N_PAR)

    lastb = row0 + (_N_BATCHES - 1) * _BATCH
    for j in range(_N_PAR):
        lastw = row0 + (_N_BATCHES - _N_PAR + j) * _BATCH
        pltpu.make_async_copy(
            outbufs[j], out_hbm.at[pl.ds(lastw, _BATCH)], osems[j]).wait()
        pltpu.make_async_copy(
            in_hbm.at[pl.ds(lastb, _BATCH)], inbufs[j], isems[j]).wait()


def kernel(input):
    mesh = plsc.VectorSubcoreMesh(core_axis_name="c", subcore_axis_name="s")
    fn = functools.partial(
        pl.kernel,
        out_type=jax.ShapeDtypeStruct((_N_ROWS, _N_CLASSES), jnp.float32),
        mesh=mesh,
        scratch_types=(
            [pltpu.VMEM((_BATCH, _N_CLASSES), jnp.float32)] * 8
            + [pltpu.SemaphoreType.DMA] * 8
        ),
        compiler_params=pltpu.CompilerParams(needs_layout_passes=False),
    )(_sc_body)
    return fn(input)
